# baseline (speedup 1.0000x reference)
import jax
import jax.numpy as jnp
from jax import lax
from jax.experimental import pallas as pl
from jax.experimental.pallas import tpu as pltpu
from jax.experimental.pallas import tpu_sc as plsc

N = 50000
NP = 51200
E = 1600000
EPAD = 1605632
EROWS = EPAD // 128
C = 10
H = 32
B = 64
NB = 1024
NBLK = NP // NB
TPT = NP // 16
HMPT = 49
HROWS = EROWS // 2
CPT = HMPT * 8

_SC_PARAMS = pltpu.CompilerParams(use_tc_tiling_on_sc=False)


def _sc_edge_body(*refs):
  tabs = refs[:2 * C]
  row2d, col2d, zeros = refs[2 * C:2 * C + 3]
  s_out, deg_out = refs[2 * C + 3:2 * C + 5]
  acc, idxr, idxc, vba, vbb, vbs = refs[2 * C + 5:2 * C + 11]
  gsem = refs[2 * C + 11:2 * C + 14]
  ssem = refs[2 * C + 14:2 * C + 16]
  isem = refs[2 * C + 16]

  cid = lax.axis_index("c")
  sid = lax.axis_index("s")
  rlo = sid * TPT
  base = cid * HROWS + sid * CPT

  def idx_fetch(m, slot):
    pltpu.async_copy(row2d.at[pl.ds(base + m * 8, 8)], idxr.at[slot], isem)
    pltpu.async_copy(col2d.at[pl.ds(base + m * 8, 8)], idxc.at[slot], isem)

  def idx_wait(slot):
    pltpu.make_async_copy(row2d.at[pl.ds(0, 8)], idxr.at[slot], isem).wait()
    pltpu.make_async_copy(col2d.at[pl.ds(0, 8)], idxc.at[slot], isem).wait()

  for c in range(C):
    tab_a = tabs[c]
    tab_b = tabs[C + c]
    pltpu.sync_copy(zeros.at[pl.ds(rlo, TPT)], acc.at[pl.ds(rlo, TPT)])
    plsc.subcore_barrier()
    idx_fetch(0, 0)

    def macro_body(m, carry, tab_a=tab_a, tab_b=tab_b):
      mb = lax.rem(m, 2)
      idx_wait(mb)
      idx_fetch(lax.min(m + 1, HMPT - 1), 1 - mb)
      pend = {}
      for j in range(3):
        ga = pltpu.async_copy(tab_a.at[idxr.at[mb, j]], vba.at[j], gsem[j])
        gb = pltpu.async_copy(tab_b.at[idxc.at[mb, j]], vbb.at[j], gsem[j])
        pend[j] = (ga, gb)
      scat = {}
      for j in range(8):
        b = j % 3
        s = j % 2
        da, db = pend.pop(j)
        da.wait()
        db.wait()
        if j >= 2:
          scat.pop(j - 2).wait()

        def cvt4(i, carry, b=b, s=s):
          r = 4 * i
          for dr in range(4):
            w = vba[b, r + dr, :] + vbb[b, r + dr, :]
            lo = jnp.bitwise_and(w, jnp.int32(65535)).astype(jnp.float32)
            hi = lax.shift_right_logical(w, 16).astype(jnp.float32)
            vbs[s, r + dr, pl.ds(0, 16)] = jnp.maximum(lo, 32768.0)
            vbs[s, r + dr, pl.ds(16, 16)] = jnp.maximum(hi, 32768.0)
          return carry

        lax.fori_loop(0, 32, cvt4, 0)
        if j + 3 <= 7:
          nb = (j + 3) % 3
          ga = pltpu.async_copy(tab_a.at[idxr.at[mb, j + 3]], vba.at[nb],
                                gsem[nb])
          gb = pltpu.async_copy(tab_b.at[idxc.at[mb, j + 3]], vbb.at[nb],
                                gsem[nb])
          pend[j + 3] = (ga, gb)
        scat[j] = pltpu.async_copy(vbs.at[s], acc.at[idxc.at[mb, j]],
                                   ssem[s], add=True)
      for k in sorted(scat):
        scat.pop(k).wait()
      return carry

    lax.fori_loop(0, HMPT, macro_body, 0)
    idx_wait(0)
    plsc.subcore_barrier()
    pltpu.sync_copy(acc.at[pl.ds(rlo, TPT)],
                    s_out.at[cid, c, pl.ds(rlo, TPT)])
    plsc.subcore_barrier()

  def ones_row(i, carry):
    vbs[0, i, pl.ds(0, 16)] = jnp.full((16,), 1.0, jnp.float32)
    vbs[0, i, pl.ds(16, 16)] = jnp.full((16,), 1.0, jnp.float32)
    return carry

  lax.fori_loop(0, 128, ones_row, 0)
  pltpu.sync_copy(zeros.at[pl.ds(rlo, TPT)], acc.at[pl.ds(rlo, TPT)])
  plsc.subcore_barrier()

  def deg_macro(m, carry):
    pltpu.sync_copy(col2d.at[pl.ds(base + m * 8, 8)], idxc.at[0])
    for j in range(8):
      pltpu.sync_copy(vbs.at[0], acc.at[idxc.at[0, j]], add=True)
    return carry

  lax.fori_loop(0, HMPT, deg_macro, 0)
  plsc.subcore_barrier()
  pltpu.sync_copy(acc.at[pl.ds(rlo, TPT)], deg_out.at[cid, pl.ds(rlo, TPT)])


def _sc_gcn_body(t0, t1, t2, t3, row2d, col2d, zeros, t_out, acc, idxr, idxc,
                 vb, *sems):
  gsem = sems[:6]
  ssem = sems[6:12]
  isem = sems[12]
  cid = lax.axis_index("c")
  sid = lax.axis_index("s")
  rlo = sid * TPT
  base = cid * HROWS + sid * CPT

  def idx_fetch(m, slot):
    pltpu.async_copy(row2d.at[pl.ds(base + m * 8, 8)], idxr.at[slot], isem)
    pltpu.async_copy(col2d.at[pl.ds(base + m * 8, 8)], idxc.at[slot], isem)

  def idx_wait(slot):
    pltpu.make_async_copy(row2d.at[pl.ds(0, 8)], idxr.at[slot], isem).wait()
    pltpu.make_async_copy(col2d.at[pl.ds(0, 8)], idxc.at[slot], isem).wait()

  for jc, tab in enumerate((t0, t1, t2, t3)):
    pltpu.sync_copy(zeros.at[pl.ds(rlo, TPT)], acc.at[pl.ds(rlo, TPT)])
    plsc.subcore_barrier()
    idx_fetch(0, 0)

    def macro_body(m, carry, tab=tab):
      mb = lax.rem(m, 2)
      idx_wait(mb)
      idx_fetch(lax.min(m + 1, HMPT - 1), 1 - mb)
      pend = {j: pltpu.async_copy(tab.at[idxr.at[mb, j]], vb.at[j], gsem[j])
              for j in range(3)}
      scat = {}
      for j in range(8):
        b = j % 6
        pend.pop(j).wait()
        scat[j] = pltpu.async_copy(vb.at[b], acc.at[idxc.at[mb, j]],
                                   ssem[b], add=True)
        if j < 5:
          nb = (j + 3) % 6
          if j >= 3:
            scat.pop(j - 3).wait()
          pend[j + 3] = pltpu.async_copy(tab.at[idxr.at[mb, j + 3]],
                                         vb.at[nb], gsem[nb])
      for k in sorted(scat):
        scat.pop(k).wait()
      return carry

    lax.fori_loop(0, HMPT, macro_body, 0)
    idx_wait(0)
    plsc.subcore_barrier()
    pltpu.sync_copy(acc.at[pl.ds(rlo, TPT)],
                    t_out.at[cid, jc, pl.ds(rlo, TPT)])
    plsc.subcore_barrier()


def _make_sc_edge():
  return pl.kernel(
      _sc_edge_body,
      out_type=[jax.ShapeDtypeStruct((2, C, NP, 32), jnp.float32),
                jax.ShapeDtypeStruct((2, NP, 32), jnp.float32)],
      mesh=plsc.VectorSubcoreMesh(core_axis_name="c", subcore_axis_name="s"),
      scratch_types=_sc_scratch(),
      compiler_params=_SC_PARAMS,
  )


def _sc_scratch():
  return [pltpu.VMEM_SHARED((NP, 32), jnp.float32),
          pltpu.VMEM((2, 8, 128), jnp.int32),
          pltpu.VMEM((2, 8, 128), jnp.int32),
          pltpu.VMEM((3, 128, 16), jnp.int32),
          pltpu.VMEM((3, 128, 16), jnp.int32),
          pltpu.VMEM((2, 128, 32), jnp.float32)] + [
              pltpu.SemaphoreType.DMA] * 6


def _sc_gcn_scratch():
  return [pltpu.VMEM_SHARED((NP, 32), jnp.float32),
          pltpu.VMEM((2, 8, 128), jnp.int32),
          pltpu.VMEM((2, 8, 128), jnp.int32),
          pltpu.VMEM((6, 128, 32), jnp.float32)] + [
              pltpu.SemaphoreType.DMA] * 13


def _make_sc_gcn():
  return pl.kernel(
      _sc_gcn_body,
      out_type=jax.ShapeDtypeStruct((2, 4, NP, 32), jnp.float32),
      mesh=plsc.VectorSubcoreMesh(core_axis_name="c", subcore_axis_name="s"),
      scratch_types=_sc_gcn_scratch(),
      compiler_params=_SC_PARAMS,
  )


def _tc1_body(x_ref, k_ref, b_ref, *o_refs):
  h = (jnp.dot(x_ref[...], k_ref[...], preferred_element_type=jnp.float32)
       + b_ref[...])
  for t in range(2 * C):
    hs = h[:, 32 * t:32 * t + 32]
    qs = (jnp.clip(jnp.round(hs * 1024.0), -8191.0, 8191.0)
          .astype(jnp.int32) + 16384)
    o_refs[t][...] = qs[:, :16] | (qs[:, 16:] << 16)


def _tc2_body(x_ref, s_ref, deg_ref, wx_ref, we_ref, vb2_ref, *o_refs):
  degv = deg_ref[0, :, 0:1] + deg_ref[1, :, 0:1] + 1.0
  h = (jnp.dot(x_ref[...], wx_ref[...], preferred_element_type=jnp.float32)
       + (degv - 1.0) * vb2_ref[...])
  for c in range(C):
    s_c = s_ref[0, c] + s_ref[1, c]
    h += jnp.dot(s_c, we_ref[c], preferred_element_type=jnp.float32)
  scaled = lax.rsqrt(degv) * h
  for t in range(4):
    o_refs[t][...] = scaled[:, 32 * t:32 * t + 32]


def _gather_t(t_ref):
  return jnp.concatenate([t_ref[0, jc] + t_ref[1, jc] for jc in range(4)],
                         axis=1)


def _tc3_body(t_ref, s0, s1, s2, s3, deg_ref, bg_ref, w_ref, *o_refs):
  degv = deg_ref[0, :, 0:1] + deg_ref[1, :, 0:1] + 1.0
  dinv = lax.rsqrt(degv)
  sc_full = jnp.concatenate([s0[...], s1[...], s2[...], s3[...]], axis=1)
  x3 = jnp.maximum(dinv * (_gather_t(t_ref) + sc_full) + bg_ref[...], 0.0)
  scaled = dinv * jnp.dot(x3, w_ref[...], preferred_element_type=jnp.float32)
  for t in range(4):
    o_refs[t][...] = scaled[:, 32 * t:32 * t + 32]


def _tc4_body(t_ref, s0, s1, s2, s3, deg_ref, bg_ref, bat_ref, wc_ref, bc_ref,
              o_ref, sums, cnts):
  i = pl.program_id(0)

  @pl.when(i == 0)
  def _():
    sums[...] = jnp.zeros_like(sums)
    cnts[...] = jnp.zeros_like(cnts)

  degv = deg_ref[0, :, 0:1] + deg_ref[1, :, 0:1] + 1.0
  dinv = lax.rsqrt(degv)
  sc_full = jnp.concatenate([s0[...], s1[...], s2[...], s3[...]], axis=1)
  x4 = jnp.maximum(dinv * (_gather_t(t_ref) + sc_full) + bg_ref[...], 0.0)
  oh = (bat_ref[...] == lax.broadcasted_iota(jnp.int32, (1, B), 1)
        ).astype(jnp.float32)
  sums[...] += lax.dot_general(oh, x4, (((0,), (0,)), ((), ())),
                               preferred_element_type=jnp.float32)
  cnts[...] += jnp.broadcast_to(jnp.sum(oh, axis=0)[:, None], (B, 128))

  @pl.when(i == NBLK - 1)
  def _():
    pooled = sums[...] / jnp.maximum(cnts[...], 1.0)
    o_ref[...] = (jnp.dot(pooled, wc_ref[...],
                          preferred_element_type=jnp.float32) + bc_ref[...])


def kernel(x, edge_index, batch, Wf, bf, W1, b1, W2, b2, Wg1, bg1, Wg2, bg2,
           Wc, bc):
  f32 = jnp.float32
  W1t, W1b = W1[:H], W1[H:]
  KA = jnp.einsum("cih,hk->cik", Wf, W1t)
  KB = jnp.einsum("cih,hk->cik", Wf, W1b)
  Kmat = jnp.concatenate([KA.transpose(1, 0, 2).reshape(3, 320),
                          KB.transpose(1, 0, 2).reshape(3, 320)], axis=1)
  biasA = (bf @ W1t).reshape(320)
  biasB = (bf @ W1b + b1[None, :]).reshape(320)
  biasv = jnp.concatenate([biasA, biasB]).reshape(1, 640)
  Wg1x = Wg1[:3]
  Wg1a = Wg1[3:].reshape(C, H, 128)
  Weff = jnp.einsum("hk,ckm->chm", W2, Wg1a)
  Weff = Weff / 1024.0
  vb2 = (jnp.einsum("h,chm->m", b2, Wg1a)
         - 32.0 * jnp.sum(Weff, axis=(0, 1))).reshape(1, 128)

  x_pad = jnp.pad(x, ((0, NP - N), (0, 0)))
  row = edge_index[0].astype(jnp.int32)
  col = edge_index[1].astype(jnp.int32)
  pad_idx = jnp.full((EPAD - E,), N, jnp.int32)
  row2d = jnp.concatenate([row, pad_idx]).reshape(EROWS, 128)
  col2d = jnp.concatenate([col, pad_idx]).reshape(EROWS, 128)
  zeros = jnp.zeros((NP, 32), f32)
  batch_pad = jnp.concatenate(
      [batch.astype(jnp.int32), jnp.full((NP - N,), B, jnp.int32)]
  ).reshape(NP, 1)

  tab_shape = jax.ShapeDtypeStruct((NP, 32), f32)
  tab_shape_pk = jax.ShapeDtypeStruct((NP, 16), jnp.int32)
  tabs = pl.pallas_call(
      _tc1_body,
      grid=(NBLK,),
      in_specs=[pl.BlockSpec((NB, 3), lambda i: (i, 0)),
                pl.BlockSpec((3, 640), lambda i: (0, 0)),
                pl.BlockSpec((1, 640), lambda i: (0, 0))],
      out_specs=[pl.BlockSpec((NB, 16), lambda i: (i, 0))] * (2 * C),
      out_shape=[tab_shape_pk] * (2 * C),
  )(x_pad, Kmat, biasv)

  S, deg2 = _make_sc_edge()(*tabs, row2d, col2d, zeros)

  deg_spec = pl.BlockSpec((2, NB, 32), lambda i: (0, i, 0))
  chunk_spec = pl.BlockSpec((NB, 32), lambda i: (i, 0))
  s1chunks = pl.pallas_call(
      _tc2_body,
      grid=(NBLK,),
      in_specs=[pl.BlockSpec((NB, 3), lambda i: (i, 0)),
                pl.BlockSpec((2, C, NB, 32), lambda i: (0, 0, i, 0)),
                deg_spec,
                pl.BlockSpec((3, 128), lambda i: (0, 0)),
                pl.BlockSpec((C, 32, 128), lambda i: (0, 0, 0)),
                pl.BlockSpec((1, 128), lambda i: (0, 0))],
      out_specs=[chunk_spec] * 4,
      out_shape=[tab_shape] * 4,
  )(x_pad, S, deg2, Wg1x, Weff, vb2)

  T1 = _make_sc_gcn()(*s1chunks, row2d, col2d, zeros)

  t_spec = pl.BlockSpec((2, 4, NB, 32), lambda i: (0, 0, i, 0))
  s2chunks = pl.pallas_call(
      _tc3_body,
      grid=(NBLK,),
      in_specs=[t_spec, chunk_spec, chunk_spec, chunk_spec, chunk_spec,
                deg_spec,
                pl.BlockSpec((1, 128), lambda i: (0, 0)),
                pl.BlockSpec((128, 128), lambda i: (0, 0))],
      out_specs=[chunk_spec] * 4,
      out_shape=[tab_shape] * 4,
  )(T1, *s1chunks, deg2, bg1.reshape(1, 128), Wg2)

  T2 = _make_sc_gcn()(*s2chunks, row2d, col2d, zeros)

  out = pl.pallas_call(
      _tc4_body,
      grid=(NBLK,),
      in_specs=[t_spec, chunk_spec, chunk_spec, chunk_spec, chunk_spec,
                deg_spec,
                pl.BlockSpec((1, 128), lambda i: (0, 0)),
                pl.BlockSpec((NB, 1), lambda i: (i, 0)),
                pl.BlockSpec((128, 10), lambda i: (0, 0)),
                pl.BlockSpec((1, 10), lambda i: (0, 0))],
      out_specs=pl.BlockSpec((B, 10), lambda i: (0, 0)),
      out_shape=jax.ShapeDtypeStruct((B, 10), f32),
      scratch_shapes=[pltpu.VMEM((B, 128), f32), pltpu.VMEM((B, 128), f32)],
  )(T2, *s2chunks, deg2, bg2.reshape(1, 128), batch_pad, Wc,
    bc.reshape(1, 10))

  return out


# speedup vs baseline: 1.1381x; 1.1381x over previous
import jax
import jax.numpy as jnp
from jax import lax
from jax.experimental import pallas as pl
from jax.experimental.pallas import tpu as pltpu
from jax.experimental.pallas import tpu_sc as plsc

N = 50000
NP = 51200
E = 1600000
EPAD = 1605632
EROWS = EPAD // 128
C = 10
H = 32
B = 64
NB = 1024
NBLK = NP // NB
TPT = NP // 16
CPM = 14
HMPT = 28
HROWS = EROWS // 2
CPT = HMPT * CPM

_SC_PARAMS = pltpu.CompilerParams(use_tc_tiling_on_sc=False)


def _sc_edge_body(*refs):
  tabs = refs[:2 * C]
  row2d, col2d, zeros = refs[2 * C:2 * C + 3]
  s_out, deg_out = refs[2 * C + 3:2 * C + 5]
  acc, idxr, idxc, vba = refs[2 * C + 5:2 * C + 9]
  gsem = refs[2 * C + 9:2 * C + 13]
  ssem = refs[2 * C + 13:2 * C + 17]
  isem = refs[2 * C + 17]

  cid = lax.axis_index("c")
  sid = lax.axis_index("s")
  rlo = sid * TPT
  base = cid * HROWS + sid * CPT

  def idx_fetch(m, slot):
    pltpu.async_copy(row2d.at[pl.ds(base + m * CPM, CPM)], idxr.at[slot],
                     isem)
    pltpu.async_copy(col2d.at[pl.ds(base + m * CPM, CPM)], idxc.at[slot],
                     isem)

  def idx_wait(slot):
    pltpu.make_async_copy(row2d.at[pl.ds(0, CPM)], idxr.at[slot],
                          isem).wait()
    pltpu.make_async_copy(col2d.at[pl.ds(0, CPM)], idxc.at[slot],
                          isem).wait()

  for c in range(C):
    tab_a = tabs[c]
    tab_b = tabs[C + c]
    pltpu.sync_copy(zeros.at[pl.ds(rlo, TPT)], acc.at[pl.ds(rlo, TPT)])
    plsc.subcore_barrier()
    idx_fetch(0, 0)

    def macro_body(m, carry, tab_a=tab_a, tab_b=tab_b):
      mb = lax.rem(m, 2)
      idx_wait(mb)
      idx_fetch(lax.min(m + 1, HMPT - 1), 1 - mb)
      penda = {j: pltpu.async_copy(tab_a.at[idxr.at[mb, j]], vba.at[j],
                                   gsem[j]) for j in range(2)}
      pendb = {}
      scat = {}
      penda.pop(0).wait()
      pendb[0] = pltpu.async_copy(tab_b.at[idxc.at[mb, 0]], vba.at[0],
                                  gsem[0], add=True)
      for j in range(CPM):
        b = j % 4
        if j + 2 <= CPM - 1:
          nb = (j + 2) % 4
          if j >= 2:
            scat.pop(j - 2).wait()
          penda[j + 2] = pltpu.async_copy(tab_a.at[idxr.at[mb, j + 2]],
                                          vba.at[nb], gsem[nb])
        if j + 1 <= CPM - 1:
          nb1 = (j + 1) % 4
          penda.pop(j + 1).wait()
          pendb[j + 1] = pltpu.async_copy(tab_b.at[idxc.at[mb, j + 1]],
                                          vba.at[nb1], gsem[nb1], add=True)
        pendb.pop(j).wait()

        def relu8(i, carry, b=b):
          r = 8 * i
          for dr in range(8):
            for h in (0, 16):
              vba[b, r + dr, pl.ds(h, 16)] = jnp.maximum(
                  vba[b, r + dr, pl.ds(h, 16)], 0.0)
          return carry

        lax.fori_loop(0, 16, relu8, 0)
        scat[j] = pltpu.async_copy(vba.at[b], acc.at[idxc.at[mb, j]],
                                   ssem[b], add=True)
      for k in sorted(scat):
        scat.pop(k).wait()
      return carry

    lax.fori_loop(0, HMPT, macro_body, 0)
    idx_wait(0)
    plsc.subcore_barrier()
    pltpu.sync_copy(acc.at[pl.ds(rlo, TPT)],
                    s_out.at[cid, c, pl.ds(rlo, TPT)])
    plsc.subcore_barrier()

  def ones_row(i, carry):
    vba[0, i, pl.ds(0, 16)] = jnp.full((16,), 1.0, jnp.float32)
    vba[0, i, pl.ds(16, 16)] = jnp.full((16,), 1.0, jnp.float32)
    return carry

  lax.fori_loop(0, 128, ones_row, 0)
  pltpu.sync_copy(zeros.at[pl.ds(rlo, TPT)], acc.at[pl.ds(rlo, TPT)])
  plsc.subcore_barrier()

  def deg_macro(m, carry):
    pltpu.sync_copy(col2d.at[pl.ds(base + m * CPM, CPM)], idxc.at[0])
    for j in range(CPM):
      pltpu.sync_copy(vba.at[0], acc.at[idxc.at[0, j]], add=True)
    return carry

  lax.fori_loop(0, HMPT, deg_macro, 0)
  plsc.subcore_barrier()
  pltpu.sync_copy(acc.at[pl.ds(rlo, TPT)], deg_out.at[cid, pl.ds(rlo, TPT)])


def _sc_gcn_body(t0, t1, t2, t3, row2d, col2d, zeros, t_out, acc, idxr, idxc,
                 vb, *sems):
  gsem = sems[:5]
  ssem = sems[5:10]
  isem = sems[10]
  cid = lax.axis_index("c")
  sid = lax.axis_index("s")
  rlo = sid * TPT
  base = cid * HROWS + sid * CPT

  def idx_fetch(m, slot):
    pltpu.async_copy(row2d.at[pl.ds(base + m * CPM, CPM)], idxr.at[slot],
                     isem)
    pltpu.async_copy(col2d.at[pl.ds(base + m * CPM, CPM)], idxc.at[slot],
                     isem)

  def idx_wait(slot):
    pltpu.make_async_copy(row2d.at[pl.ds(0, CPM)], idxr.at[slot],
                          isem).wait()
    pltpu.make_async_copy(col2d.at[pl.ds(0, CPM)], idxc.at[slot],
                          isem).wait()

  for jc, tab in enumerate((t0, t1, t2, t3)):
    pltpu.sync_copy(zeros.at[pl.ds(rlo, TPT)], acc.at[pl.ds(rlo, TPT)])
    plsc.subcore_barrier()
    idx_fetch(0, 0)

    def macro_body(m, carry, tab=tab):
      mb = lax.rem(m, 2)
      idx_wait(mb)
      idx_fetch(lax.min(m + 1, HMPT - 1), 1 - mb)
      pend = {j: pltpu.async_copy(tab.at[idxr.at[mb, j]], vb.at[j], gsem[j])
              for j in range(3)}
      scat = {}
      for j in range(CPM):
        b = j % 5
        pend.pop(j).wait()
        scat[j] = pltpu.async_copy(vb.at[b], acc.at[idxc.at[mb, j]],
                                   ssem[b], add=True)
        if j < CPM - 3:
          nb = (j + 3) % 5
          if j >= 2:
            scat.pop(j - 2).wait()
          pend[j + 3] = pltpu.async_copy(tab.at[idxr.at[mb, j + 3]],
                                         vb.at[nb], gsem[nb])
      for k in sorted(scat):
        scat.pop(k).wait()
      return carry

    lax.fori_loop(0, HMPT, macro_body, 0)
    idx_wait(0)
    plsc.subcore_barrier()
    pltpu.sync_copy(acc.at[pl.ds(rlo, TPT)],
                    t_out.at[cid, jc, pl.ds(rlo, TPT)])
    plsc.subcore_barrier()


def _make_sc_edge():
  return pl.kernel(
      _sc_edge_body,
      out_type=[jax.ShapeDtypeStruct((2, C, NP, 32), jnp.float32),
                jax.ShapeDtypeStruct((2, NP, 32), jnp.float32)],
      mesh=plsc.VectorSubcoreMesh(core_axis_name="c", subcore_axis_name="s"),
      scratch_types=_sc_scratch(),
      compiler_params=_SC_PARAMS,
  )


def _sc_scratch():
  return [pltpu.VMEM_SHARED((NP, 32), jnp.float32),
          pltpu.VMEM((2, CPM, 128), jnp.int32),
          pltpu.VMEM((2, CPM, 128), jnp.int32),
          pltpu.VMEM((4, 128, 32), jnp.float32)] + [
              pltpu.SemaphoreType.DMA] * 9


def _sc_gcn_scratch():
  return [pltpu.VMEM_SHARED((NP, 32), jnp.float32),
          pltpu.VMEM((2, CPM, 128), jnp.int32),
          pltpu.VMEM((2, CPM, 128), jnp.int32),
          pltpu.VMEM((5, 128, 32), jnp.float32)] + [
              pltpu.SemaphoreType.DMA] * 11


def _make_sc_gcn():
  return pl.kernel(
      _sc_gcn_body,
      out_type=jax.ShapeDtypeStruct((2, 4, NP, 32), jnp.float32),
      mesh=plsc.VectorSubcoreMesh(core_axis_name="c", subcore_axis_name="s"),
      scratch_types=_sc_gcn_scratch(),
      compiler_params=_SC_PARAMS,
  )


def _tc1_body(x_ref, k_ref, b_ref, *o_refs):
  h = (jnp.dot(x_ref[...], k_ref[...], preferred_element_type=jnp.float32)
       + b_ref[...])
  for t in range(2 * C):
    o_refs[t][...] = h[:, 32 * t:32 * t + 32]


def _tc2_body(x_ref, s_ref, deg_ref, wx_ref, we_ref, vb2_ref, *o_refs):
  degv = deg_ref[0, :, 0:1] + deg_ref[1, :, 0:1] + 1.0
  h = (jnp.dot(x_ref[...], wx_ref[...], preferred_element_type=jnp.float32)
       + (degv - 1.0) * vb2_ref[...])
  for c in range(C):
    s_c = s_ref[0, c] + s_ref[1, c]
    h += jnp.dot(s_c, we_ref[c], preferred_element_type=jnp.float32)
  scaled = lax.rsqrt(degv) * h
  for t in range(4):
    o_refs[t][...] = scaled[:, 32 * t:32 * t + 32]


def _gather_t(t_ref):
  return jnp.concatenate([t_ref[0, jc] + t_ref[1, jc] for jc in range(4)],
                         axis=1)


def _tc3_body(t_ref, s0, s1, s2, s3, deg_ref, bg_ref, w_ref, *o_refs):
  degv = deg_ref[0, :, 0:1] + deg_ref[1, :, 0:1] + 1.0
  dinv = lax.rsqrt(degv)
  sc_full = jnp.concatenate([s0[...], s1[...], s2[...], s3[...]], axis=1)
  x3 = jnp.maximum(dinv * (_gather_t(t_ref) + sc_full) + bg_ref[...], 0.0)
  scaled = dinv * jnp.dot(x3, w_ref[...], preferred_element_type=jnp.float32)
  for t in range(4):
    o_refs[t][...] = scaled[:, 32 * t:32 * t + 32]


def _tc4_body(t_ref, s0, s1, s2, s3, deg_ref, bg_ref, bat_ref, wc_ref, bc_ref,
              o_ref, sums, cnts):
  i = pl.program_id(0)

  @pl.when(i == 0)
  def _():
    sums[...] = jnp.zeros_like(sums)
    cnts[...] = jnp.zeros_like(cnts)

  degv = deg_ref[0, :, 0:1] + deg_ref[1, :, 0:1] + 1.0
  dinv = lax.rsqrt(degv)
  sc_full = jnp.concatenate([s0[...], s1[...], s2[...], s3[...]], axis=1)
  x4 = jnp.maximum(dinv * (_gather_t(t_ref) + sc_full) + bg_ref[...], 0.0)
  oh = (bat_ref[...] == lax.broadcasted_iota(jnp.int32, (1, B), 1)
        ).astype(jnp.float32)
  sums[...] += lax.dot_general(oh, x4, (((0,), (0,)), ((), ())),
                               preferred_element_type=jnp.float32)
  cnts[...] += jnp.broadcast_to(jnp.sum(oh, axis=0)[:, None], (B, 128))

  @pl.when(i == NBLK - 1)
  def _():
    pooled = sums[...] / jnp.maximum(cnts[...], 1.0)
    o_ref[...] = (jnp.dot(pooled, wc_ref[...],
                          preferred_element_type=jnp.float32) + bc_ref[...])


def kernel(x, edge_index, batch, Wf, bf, W1, b1, W2, b2, Wg1, bg1, Wg2, bg2,
           Wc, bc):
  f32 = jnp.float32
  W1t, W1b = W1[:H], W1[H:]
  KA = jnp.einsum("cih,hk->cik", Wf, W1t)
  KB = jnp.einsum("cih,hk->cik", Wf, W1b)
  Kmat = jnp.concatenate([KA.transpose(1, 0, 2).reshape(3, 320),
                          KB.transpose(1, 0, 2).reshape(3, 320)], axis=1)
  biasA = (bf @ W1t).reshape(320)
  biasB = (bf @ W1b + b1[None, :]).reshape(320)
  biasv = jnp.concatenate([biasA, biasB]).reshape(1, 640)
  Wg1x = Wg1[:3]
  Wg1a = Wg1[3:].reshape(C, H, 128)
  Weff = jnp.einsum("hk,ckm->chm", W2, Wg1a)
  vb2 = jnp.einsum("h,chm->m", b2, Wg1a).reshape(1, 128)

  x_pad = jnp.pad(x, ((0, NP - N), (0, 0)))
  row = edge_index[0].astype(jnp.int32)
  col = edge_index[1].astype(jnp.int32)
  pad_idx = jnp.full((EPAD - E,), N, jnp.int32)
  row2d = jnp.concatenate([row, pad_idx]).reshape(EROWS, 128)
  col2d = jnp.concatenate([col, pad_idx]).reshape(EROWS, 128)
  zeros = jnp.zeros((NP, 32), f32)
  batch_pad = jnp.concatenate(
      [batch.astype(jnp.int32), jnp.full((NP - N,), B, jnp.int32)]
  ).reshape(NP, 1)

  tab_shape = jax.ShapeDtypeStruct((NP, 32), f32)
  tabs = pl.pallas_call(
      _tc1_body,
      grid=(NBLK,),
      in_specs=[pl.BlockSpec((NB, 3), lambda i: (i, 0)),
                pl.BlockSpec((3, 640), lambda i: (0, 0)),
                pl.BlockSpec((1, 640), lambda i: (0, 0))],
      out_specs=[pl.BlockSpec((NB, 32), lambda i: (i, 0))] * (2 * C),
      out_shape=[tab_shape] * (2 * C),
  )(x_pad, Kmat, biasv)

  S, deg2 = _make_sc_edge()(*tabs, row2d, col2d, zeros)

  deg_spec = pl.BlockSpec((2, NB, 32), lambda i: (0, i, 0))
  chunk_spec = pl.BlockSpec((NB, 32), lambda i: (i, 0))
  s1chunks = pl.pallas_call(
      _tc2_body,
      grid=(NBLK,),
      in_specs=[pl.BlockSpec((NB, 3), lambda i: (i, 0)),
                pl.BlockSpec((2, C, NB, 32), lambda i: (0, 0, i, 0)),
                deg_spec,
                pl.BlockSpec((3, 128), lambda i: (0, 0)),
                pl.BlockSpec((C, 32, 128), lambda i: (0, 0, 0)),
                pl.BlockSpec((1, 128), lambda i: (0, 0))],
      out_specs=[chunk_spec] * 4,
      out_shape=[tab_shape] * 4,
  )(x_pad, S, deg2, Wg1x, Weff, vb2)

  T1 = _make_sc_gcn()(*s1chunks, row2d, col2d, zeros)

  t_spec = pl.BlockSpec((2, 4, NB, 32), lambda i: (0, 0, i, 0))
  s2chunks = pl.pallas_call(
      _tc3_body,
      grid=(NBLK,),
      in_specs=[t_spec, chunk_spec, chunk_spec, chunk_spec, chunk_spec,
                deg_spec,
                pl.BlockSpec((1, 128), lambda i: (0, 0)),
                pl.BlockSpec((128, 128), lambda i: (0, 0))],
      out_specs=[chunk_spec] * 4,
      out_shape=[tab_shape] * 4,
  )(T1, *s1chunks, deg2, bg1.reshape(1, 128), Wg2)

  T2 = _make_sc_gcn()(*s2chunks, row2d, col2d, zeros)

  out = pl.pallas_call(
      _tc4_body,
      grid=(NBLK,),
      in_specs=[t_spec, chunk_spec, chunk_spec, chunk_spec, chunk_spec,
                deg_spec,
                pl.BlockSpec((1, 128), lambda i: (0, 0)),
                pl.BlockSpec((NB, 1), lambda i: (i, 0)),
                pl.BlockSpec((128, 10), lambda i: (0, 0)),
                pl.BlockSpec((1, 10), lambda i: (0, 0))],
      out_specs=pl.BlockSpec((B, 10), lambda i: (0, 0)),
      out_shape=jax.ShapeDtypeStruct((B, 10), f32),
      scratch_shapes=[pltpu.VMEM((B, 128), f32), pltpu.VMEM((B, 128), f32)],
  )(T2, *s2chunks, deg2, bg2.reshape(1, 128), batch_pad, Wc,
    bc.reshape(1, 10))

  return out


# speedup vs baseline: 1.3498x; 1.1860x over previous
import jax
import jax.numpy as jnp
from jax import lax
from jax.experimental import pallas as pl
from jax.experimental.pallas import tpu as pltpu
from jax.experimental.pallas import tpu_sc as plsc

N = 50000
NP = 51200
E = 1600000
EPAD = 1605632
EROWS = EPAD // 128
C = 10
H = 32
B = 64
NB = 1024
NBLK = NP // NB
TPT = NP // 16
CPM = 14
HMPT = 28
FMPT = 56
HROWS = EROWS // 2
CPT = HMPT * CPM

_SC_PARAMS = pltpu.CompilerParams(use_tc_tiling_on_sc=False)


def _sc_edge_body(*refs):
  tabs = refs[:2 * C]
  row2d, col2d, zeros = refs[2 * C:2 * C + 3]
  s_out, deg_out = refs[2 * C + 3:2 * C + 5]
  acc, idxr, idxc, vba = refs[2 * C + 5:2 * C + 9]
  gsem = refs[2 * C + 9:2 * C + 13]
  ssem = refs[2 * C + 13:2 * C + 17]
  isem = refs[2 * C + 17]

  cid = lax.axis_index("c")
  sid = lax.axis_index("s")
  rlo = sid * TPT
  fbase = sid * (FMPT * CPM)
  dbase = cid * HROWS + sid * CPT

  def idx_fetch(m, slot, base):
    pltpu.async_copy(row2d.at[pl.ds(base + m * CPM, CPM)], idxr.at[slot],
                     isem)
    pltpu.async_copy(col2d.at[pl.ds(base + m * CPM, CPM)], idxc.at[slot],
                     isem)

  def idx_wait(slot):
    pltpu.make_async_copy(row2d.at[pl.ds(0, CPM)], idxr.at[slot],
                          isem).wait()
    pltpu.make_async_copy(col2d.at[pl.ds(0, CPM)], idxc.at[slot],
                          isem).wait()

  for c in range(C):
    rounds_core = 0 if c < C // 2 else 1
    tab_a = tabs[c]
    tab_b = tabs[C + c]

    @pl.when(cid == rounds_core)
    def _(tab_a=tab_a, tab_b=tab_b, c=c):
      pltpu.sync_copy(zeros.at[pl.ds(rlo, TPT)], acc.at[pl.ds(rlo, TPT)])
      plsc.subcore_barrier()
      idx_fetch(0, 0, fbase)

      def macro_body(m, carry, tab_a=tab_a, tab_b=tab_b):
        mb = lax.rem(m, 2)
        idx_wait(mb)
        idx_fetch(lax.min(m + 1, FMPT - 1), 1 - mb, fbase)
        penda = {j: pltpu.async_copy(tab_a.at[idxr.at[mb, j]], vba.at[j],
                                     gsem[j]) for j in range(2)}
        pendb = {}
        scat = {}
        penda.pop(0).wait()
        pendb[0] = pltpu.async_copy(tab_b.at[idxc.at[mb, 0]], vba.at[0],
                                    gsem[0], add=True)
        for j in range(CPM):
          b = j % 4
          if j + 2 <= CPM - 1:
            nb = (j + 2) % 4
            if j >= 2:
              scat.pop(j - 2).wait()
            penda[j + 2] = pltpu.async_copy(tab_a.at[idxr.at[mb, j + 2]],
                                            vba.at[nb], gsem[nb])
          if j + 1 <= CPM - 1:
            nb1 = (j + 1) % 4
            penda.pop(j + 1).wait()
            pendb[j + 1] = pltpu.async_copy(tab_b.at[idxc.at[mb, j + 1]],
                                            vba.at[nb1], gsem[nb1], add=True)
          pendb.pop(j).wait()

          def relu8(i, carry, b=b):
            r = 8 * i
            for dr in range(8):
              for h in (0, 16):
                vba[b, r + dr, pl.ds(h, 16)] = jnp.maximum(
                    vba[b, r + dr, pl.ds(h, 16)], 0.0)
            return carry

          lax.fori_loop(0, 16, relu8, 0)
          scat[j] = pltpu.async_copy(vba.at[b], acc.at[idxc.at[mb, j]],
                                     ssem[b], add=True)
        for k in sorted(scat):
          scat.pop(k).wait()
        return carry

      lax.fori_loop(0, FMPT, macro_body, 0)
      idx_wait(0)
      plsc.subcore_barrier()
      pltpu.sync_copy(acc.at[pl.ds(rlo, TPT)],
                      s_out.at[c, pl.ds(rlo, TPT)])
      plsc.subcore_barrier()

  def ones_row(i, carry):
    vba[0, i, pl.ds(0, 16)] = jnp.full((16,), 1.0, jnp.float32)
    vba[0, i, pl.ds(16, 16)] = jnp.full((16,), 1.0, jnp.float32)
    return carry

  lax.fori_loop(0, 128, ones_row, 0)
  pltpu.sync_copy(zeros.at[pl.ds(rlo, TPT)], acc.at[pl.ds(rlo, TPT)])
  plsc.subcore_barrier()

  def deg_macro(m, carry):
    pltpu.sync_copy(col2d.at[pl.ds(dbase + m * CPM, CPM)], idxc.at[0])
    for j in range(CPM):
      pltpu.sync_copy(vba.at[0], acc.at[idxc.at[0, j]], add=True)
    return carry

  lax.fori_loop(0, HMPT, deg_macro, 0)
  plsc.subcore_barrier()
  pltpu.sync_copy(acc.at[pl.ds(rlo, TPT)], deg_out.at[cid, pl.ds(rlo, TPT)])


def _sc_gcn_body(t0, t1, t2, t3, row2d, col2d, zeros, t_out, acc, idxr, idxc,
                 vb, *sems):
  gsem = sems[:5]
  ssem = sems[5:10]
  isem = sems[10]
  cid = lax.axis_index("c")
  sid = lax.axis_index("s")
  rlo = sid * TPT
  fbase = sid * (FMPT * CPM)

  def idx_fetch(m, slot):
    pltpu.async_copy(row2d.at[pl.ds(fbase + m * CPM, CPM)], idxr.at[slot],
                     isem)
    pltpu.async_copy(col2d.at[pl.ds(fbase + m * CPM, CPM)], idxc.at[slot],
                     isem)

  def idx_wait(slot):
    pltpu.make_async_copy(row2d.at[pl.ds(0, CPM)], idxr.at[slot],
                          isem).wait()
    pltpu.make_async_copy(col2d.at[pl.ds(0, CPM)], idxc.at[slot],
                          isem).wait()

  for jc, tab in enumerate((t0, t1, t2, t3)):

    @pl.when(cid == jc // 2)
    def _(tab=tab, jc=jc):
      pltpu.sync_copy(zeros.at[pl.ds(rlo, TPT)], acc.at[pl.ds(rlo, TPT)])
      plsc.subcore_barrier()
      idx_fetch(0, 0)

      def macro_body(m, carry, tab=tab):
        mb = lax.rem(m, 2)
        idx_wait(mb)
        idx_fetch(lax.min(m + 1, FMPT - 1), 1 - mb)
        pend = {j: pltpu.async_copy(tab.at[idxr.at[mb, j]], vb.at[j],
                                    gsem[j]) for j in range(3)}
        scat = {}
        for j in range(CPM):
          b = j % 5
          pend.pop(j).wait()
          scat[j] = pltpu.async_copy(vb.at[b], acc.at[idxc.at[mb, j]],
                                     ssem[b], add=True)
          if j < CPM - 3:
            nb = (j + 3) % 5
            if j >= 2:
              scat.pop(j - 2).wait()
            pend[j + 3] = pltpu.async_copy(tab.at[idxr.at[mb, j + 3]],
                                           vb.at[nb], gsem[nb])
        for k in sorted(scat):
          scat.pop(k).wait()
        return carry

      lax.fori_loop(0, FMPT, macro_body, 0)
      idx_wait(0)
      plsc.subcore_barrier()
      pltpu.sync_copy(acc.at[pl.ds(rlo, TPT)],
                      t_out.at[jc, pl.ds(rlo, TPT)])
      plsc.subcore_barrier()


def _make_sc_edge():
  return pl.kernel(
      _sc_edge_body,
      out_type=[jax.ShapeDtypeStruct((C, NP, 32), jnp.float32),
                jax.ShapeDtypeStruct((2, NP, 32), jnp.float32)],
      mesh=plsc.VectorSubcoreMesh(core_axis_name="c", subcore_axis_name="s"),
      scratch_types=_sc_scratch(),
      compiler_params=_SC_PARAMS,
  )


def _sc_scratch():
  return [pltpu.VMEM_SHARED((NP, 32), jnp.float32),
          pltpu.VMEM((2, CPM, 128), jnp.int32),
          pltpu.VMEM((2, CPM, 128), jnp.int32),
          pltpu.VMEM((4, 128, 32), jnp.float32)] + [
              pltpu.SemaphoreType.DMA] * 9


def _sc_gcn_scratch():
  return [pltpu.VMEM_SHARED((NP, 32), jnp.float32),
          pltpu.VMEM((2, CPM, 128), jnp.int32),
          pltpu.VMEM((2, CPM, 128), jnp.int32),
          pltpu.VMEM((5, 128, 32), jnp.float32)] + [
              pltpu.SemaphoreType.DMA] * 11


def _make_sc_gcn():
  return pl.kernel(
      _sc_gcn_body,
      out_type=jax.ShapeDtypeStruct((4, NP, 32), jnp.float32),
      mesh=plsc.VectorSubcoreMesh(core_axis_name="c", subcore_axis_name="s"),
      scratch_types=_sc_gcn_scratch(),
      compiler_params=_SC_PARAMS,
  )


def _tc1_body(x_ref, k_ref, b_ref, *o_refs):
  h = (jnp.dot(x_ref[...], k_ref[...], preferred_element_type=jnp.float32)
       + b_ref[...])
  for t in range(2 * C):
    o_refs[t][...] = h[:, 32 * t:32 * t + 32]


def _tc2_body(x_ref, s_ref, deg_ref, wx_ref, we_ref, vb2_ref, *o_refs):
  degv = deg_ref[0, :, 0:1] + deg_ref[1, :, 0:1] + 1.0
  h = (jnp.dot(x_ref[...], wx_ref[...], preferred_element_type=jnp.float32)
       + (degv - 1.0) * vb2_ref[...])
  for c in range(C):
    h += jnp.dot(s_ref[c], we_ref[c], preferred_element_type=jnp.float32)
  scaled = lax.rsqrt(degv) * h
  for t in range(4):
    o_refs[t][...] = scaled[:, 32 * t:32 * t + 32]


def _gather_t(t_ref):
  return jnp.concatenate([t_ref[jc] for jc in range(4)], axis=1)


def _tc3_body(t_ref, s0, s1, s2, s3, deg_ref, bg_ref, w_ref, *o_refs):
  degv = deg_ref[0, :, 0:1] + deg_ref[1, :, 0:1] + 1.0
  dinv = lax.rsqrt(degv)
  sc_full = jnp.concatenate([s0[...], s1[...], s2[...], s3[...]], axis=1)
  x3 = jnp.maximum(dinv * (_gather_t(t_ref) + sc_full) + bg_ref[...], 0.0)
  scaled = dinv * jnp.dot(x3, w_ref[...], preferred_element_type=jnp.float32)
  for t in range(4):
    o_refs[t][...] = scaled[:, 32 * t:32 * t + 32]


def _tc4_body(t_ref, s0, s1, s2, s3, deg_ref, bg_ref, bat_ref, wc_ref, bc_ref,
              o_ref, sums, cnts):
  i = pl.program_id(0)

  @pl.when(i == 0)
  def _():
    sums[...] = jnp.zeros_like(sums)
    cnts[...] = jnp.zeros_like(cnts)

  degv = deg_ref[0, :, 0:1] + deg_ref[1, :, 0:1] + 1.0
  dinv = lax.rsqrt(degv)
  sc_full = jnp.concatenate([s0[...], s1[...], s2[...], s3[...]], axis=1)
  x4 = jnp.maximum(dinv * (_gather_t(t_ref) + sc_full) + bg_ref[...], 0.0)
  oh = (bat_ref[...] == lax.broadcasted_iota(jnp.int32, (1, B), 1)
        ).astype(jnp.float32)
  sums[...] += lax.dot_general(oh, x4, (((0,), (0,)), ((), ())),
                               preferred_element_type=jnp.float32)
  cnts[...] += jnp.broadcast_to(jnp.sum(oh, axis=0)[:, None], (B, 128))

  @pl.when(i == NBLK - 1)
  def _():
    pooled = sums[...] / jnp.maximum(cnts[...], 1.0)
    o_ref[...] = (jnp.dot(pooled, wc_ref[...],
                          preferred_element_type=jnp.float32) + bc_ref[...])


def kernel(x, edge_index, batch, Wf, bf, W1, b1, W2, b2, Wg1, bg1, Wg2, bg2,
           Wc, bc):
  f32 = jnp.float32
  W1t, W1b = W1[:H], W1[H:]
  KA = jnp.einsum("cih,hk->cik", Wf, W1t)
  KB = jnp.einsum("cih,hk->cik", Wf, W1b)
  Kmat = jnp.concatenate([KA.transpose(1, 0, 2).reshape(3, 320),
                          KB.transpose(1, 0, 2).reshape(3, 320)], axis=1)
  biasA = (bf @ W1t).reshape(320)
  biasB = (bf @ W1b + b1[None, :]).reshape(320)
  biasv = jnp.concatenate([biasA, biasB]).reshape(1, 640)
  Wg1x = Wg1[:3]
  Wg1a = Wg1[3:].reshape(C, H, 128)
  Weff = jnp.einsum("hk,ckm->chm", W2, Wg1a)
  vb2 = jnp.einsum("h,chm->m", b2, Wg1a).reshape(1, 128)

  x_pad = jnp.pad(x, ((0, NP - N), (0, 0)))
  row = edge_index[0].astype(jnp.int32)
  col = edge_index[1].astype(jnp.int32)
  pad_idx = jnp.full((EPAD - E,), N, jnp.int32)
  row2d = jnp.concatenate([row, pad_idx]).reshape(EROWS, 128)
  col2d = jnp.concatenate([col, pad_idx]).reshape(EROWS, 128)
  zeros = jnp.zeros((NP, 32), f32)
  batch_pad = jnp.concatenate(
      [batch.astype(jnp.int32), jnp.full((NP - N,), B, jnp.int32)]
  ).reshape(NP, 1)

  tab_shape = jax.ShapeDtypeStruct((NP, 32), f32)
  tabs = pl.pallas_call(
      _tc1_body,
      grid=(NBLK,),
      in_specs=[pl.BlockSpec((NB, 3), lambda i: (i, 0)),
                pl.BlockSpec((3, 640), lambda i: (0, 0)),
                pl.BlockSpec((1, 640), lambda i: (0, 0))],
      out_specs=[pl.BlockSpec((NB, 32), lambda i: (i, 0))] * (2 * C),
      out_shape=[tab_shape] * (2 * C),
  )(x_pad, Kmat, biasv)

  S, deg2 = _make_sc_edge()(*tabs, row2d, col2d, zeros)

  deg_spec = pl.BlockSpec((2, NB, 32), lambda i: (0, i, 0))
  chunk_spec = pl.BlockSpec((NB, 32), lambda i: (i, 0))
  s1chunks = pl.pallas_call(
      _tc2_body,
      grid=(NBLK,),
      in_specs=[pl.BlockSpec((NB, 3), lambda i: (i, 0)),
                pl.BlockSpec((C, NB, 32), lambda i: (0, i, 0)),
                deg_spec,
                pl.BlockSpec((3, 128), lambda i: (0, 0)),
                pl.BlockSpec((C, 32, 128), lambda i: (0, 0, 0)),
                pl.BlockSpec((1, 128), lambda i: (0, 0))],
      out_specs=[chunk_spec] * 4,
      out_shape=[tab_shape] * 4,
  )(x_pad, S, deg2, Wg1x, Weff, vb2)

  T1 = _make_sc_gcn()(*s1chunks, row2d, col2d, zeros)

  t_spec = pl.BlockSpec((4, NB, 32), lambda i: (0, i, 0))
  s2chunks = pl.pallas_call(
      _tc3_body,
      grid=(NBLK,),
      in_specs=[t_spec, chunk_spec, chunk_spec, chunk_spec, chunk_spec,
                deg_spec,
                pl.BlockSpec((1, 128), lambda i: (0, 0)),
                pl.BlockSpec((128, 128), lambda i: (0, 0))],
      out_specs=[chunk_spec] * 4,
      out_shape=[tab_shape] * 4,
  )(T1, *s1chunks, deg2, bg1.reshape(1, 128), Wg2)

  T2 = _make_sc_gcn()(*s2chunks, row2d, col2d, zeros)

  out = pl.pallas_call(
      _tc4_body,
      grid=(NBLK,),
      in_specs=[t_spec, chunk_spec, chunk_spec, chunk_spec, chunk_spec,
                deg_spec,
                pl.BlockSpec((1, 128), lambda i: (0, 0)),
                pl.BlockSpec((NB, 1), lambda i: (i, 0)),
                pl.BlockSpec((128, 10), lambda i: (0, 0)),
                pl.BlockSpec((1, 10), lambda i: (0, 0))],
      out_specs=pl.BlockSpec((B, 10), lambda i: (0, 0)),
      out_shape=jax.ShapeDtypeStruct((B, 10), f32),
      scratch_shapes=[pltpu.VMEM((B, 128), f32), pltpu.VMEM((B, 128), f32)],
  )(T2, *s2chunks, deg2, bg2.reshape(1, 128), batch_pad, Wc,
    bc.reshape(1, 10))

  return out
